# Initial kernel scaffold; baseline (speedup 1.0000x reference)
import jax
import jax.numpy as jnp
from jax.experimental import pallas as pl


def kernel(edge_index, embeddings, Ws, a_src, a_dst, bias):
    raise NotImplementedError("write your pallas kernel here")


# speedup vs baseline: 8.9371x; 8.9371x over previous
import jax
import jax.numpy as jnp
from jax import lax
from jax.experimental import pallas as pl
from jax.experimental.pallas import tpu as pltpu
from jax.experimental.pallas import tpu_sc as plsc

N = 10000
H = 128
E = 160000
F = 2 * E

NC, NS, L = 2, 16, 16
NW = NC * NS

NP = 10240
UQ = 10496
TRASH = 10240

E_TILE = E // NW
CH = 128
NCHUNK = 40
EP_TILE = NCHUNK * CH
E_LAST = E_TILE - (NCHUNK - 1) * CH

HV = 20000
HROWS = HV // CH
HREM = HV - HROWS * CH

_MESH = plsc.VectorSubcoreMesh(core_axis_name="c", subcore_axis_name="s",
                               num_cores=NC, num_subcores=NS)

_i32 = jnp.int32
_f32 = jnp.float32


def _iota16():
    return lax.iota(_i32, L)


def _leaky(x):
    return jnp.where(x > 0, x, 0.2 * x)


def _splat_lane(vec, r):
    return jnp.broadcast_to(lax.slice(vec, (r,), (r + 1,)), (L,))


def _pre_body(flat, emb, inv_out, x0_out,
              vals_v, ones_v, cnt_v, rank_v, evals_v, inv_v,
              idx_u, val_u, zb_v, idxrow_v, rows_v,
              cnt_sh, uniq_sh, sem):
    s = lax.axis_index("s")
    c = lax.axis_index("c")
    wid = c * NS + s

    for j in range(656 // L):
        zb_v[pl.ds(j * L, L)] = jnp.zeros((L,), _i32)
    for j in range(CH // L):
        ones_v[pl.ds(j * L, L)] = jnp.ones((L,), _i32)
    pltpu.sync_copy(zb_v, cnt_sh.at[pl.ds(s * 656, 656)])
    pltpu.sync_copy(zb_v, uniq_sh.at[pl.ds(s * 656, 656)])
    for j in range(CH // L):
        off = HREM + j * L
        if off < CH:
            vals_v[HROWS, pl.ds(off, L)] = jnp.full((L,), TRASH, _i32)
    plsc.subcore_barrier()

    def _hist_load(j, _):
        pltpu.sync_copy(flat.at[pl.ds(s * HV + j * CH, CH)], vals_v.at[j])
        return 0

    lax.fori_loop(0, HROWS, _hist_load, 0)
    pltpu.sync_copy(flat.at[pl.ds(s * HV + HROWS * CH, HREM)],
                    vals_v.at[HROWS, pl.ds(0, HREM)])

    def _hist_scatter(j, _):
        pltpu.sync_copy(ones_v, cnt_sh.at[vals_v.at[j]], add=True)
        return 0

    lax.fori_loop(0, HROWS + 1, _hist_scatter, 0)
    plsc.subcore_barrier()

    pltpu.sync_copy(cnt_sh.at[pl.ds(0, NP)], cnt_v)

    def _scan_step(i, carry):
        v = cnt_v[pl.ds(i * L, L)]
        b = jnp.where(v > 0, 1, 0).astype(_i32)
        ps = plsc.cumsum(b) + carry
        rank_v[pl.ds(i * L, L)] = ps
        return jnp.max(ps)

    lax.fori_loop(0, NP // L, _scan_step, jnp.int32(0))

    pltpu.sync_copy(flat.at[pl.ds(wid * 10000, 10000)], evals_v)

    def _inv_step(i, _):
        idx = evals_v[pl.ds(i * L, L)]
        inv_v[pl.ds(i * L, L)] = plsc.load_gather(rank_v, [idx]) - 1
        return 0

    lax.fori_loop(0, 10000 // L, _inv_step, 0)
    pltpu.sync_copy(inv_v, inv_out.at[pl.ds(wid * 10000, 10000)])

    base = s * 640
    for i in range(640 // L):
        off = base + i * L
        vv = off + _iota16()
        cntv = cnt_v[pl.ds(off, L)]
        rankv = rank_v[pl.ds(off, L)]
        tgt = jnp.where(cntv > 0, rankv - 1, TRASH)
        idx_u[i // 8, pl.ds((i % 8) * L, L)] = tgt
        val_u[i // 8, pl.ds((i % 8) * L, L)] = vv
    for j in range(5):
        pltpu.sync_copy(val_u.at[j], uniq_sh.at[idx_u.at[j]])
    plsc.subcore_barrier()

    r0 = wid * 320
    for j in range(4):
        pltpu.sync_copy(uniq_sh.at[pl.ds(r0 + j * 80, 80)], idxrow_v.at[j])
        pltpu.async_copy(emb.at[idxrow_v.at[j]],
                         rows_v.at[pl.ds(j * 80, 80)], sem).wait()
    pltpu.sync_copy(rows_v, x0_out.at[pl.ds(r0, 320)])


_pre = pl.kernel(
    _pre_body,
    out_type=(jax.ShapeDtypeStruct((F,), _i32),
              jax.ShapeDtypeStruct((NP, H), _f32)),
    mesh=_MESH,
    scratch_types=[
        pltpu.VMEM((HROWS + 1, CH), _i32),
        pltpu.VMEM((CH,), _i32),
        pltpu.VMEM((NP,), _i32),
        pltpu.VMEM((NP,), _i32),
        pltpu.VMEM((10000,), _i32),
        pltpu.VMEM((10000,), _i32),
        pltpu.VMEM((5, CH), _i32),
        pltpu.VMEM((5, CH), _i32),
        pltpu.VMEM((656,), _i32),
        pltpu.VMEM((4, 80), _i32),
        pltpu.VMEM((320, H), _f32),
        pltpu.VMEM_SHARED((UQ,), _i32),
        pltpu.VMEM_SHARED((UQ,), _i32),
        pltpu.SemaphoreType.DMA,
    ],
    compiler_params=pltpu.CompilerParams(needs_layout_passes=False, use_tc_tiling_on_sc=False),
)


HH = H // 2


def _edge_body(src, dst, hs, hd, h0, h1, nump, denp,
               hs_v, hd_v, srcl_v, dstl_v, ex_v, src2_v, dst2_v,
               rows_a, rows_b, zb2, zb1,
               num_sh, den_sh, sem_a, sem_b):
    s = lax.axis_index("s")
    c = lax.axis_index("c")
    wid = c * NS + s
    base_e = wid * E_TILE

    for r in range(L):
        for j in range(HH // L):
            zb2[r, pl.ds(j * L, L)] = jnp.zeros((L,), _f32)
    for j in range(640 // L):
        zb1[pl.ds(j * L, L)] = jnp.zeros((L,), _f32)

    def _zero_num(t, _):
        pltpu.sync_copy(zb2, num_sh.at[pl.ds(s * 640 + t * L, L)])
        return 0

    lax.fori_loop(0, 640 // L, _zero_num, 0)
    pltpu.sync_copy(zb1, den_sh.at[pl.ds(s * 640, 640)])

    for j in range(8):
        srcl_v[pl.ds(E_TILE - 8 + j * L, L)] = jnp.zeros((L,), _i32)
        dstl_v[pl.ds(E_TILE - 8 + j * L, L)] = jnp.zeros((L,), _i32)
    for j in range(CH // L):
        src2_v[NCHUNK - 1, pl.ds(j * L, L)] = jnp.zeros((L,), _i32)
        dst2_v[NCHUNK - 1, pl.ds(j * L, L)] = jnp.zeros((L,), _i32)
    pltpu.sync_copy(src.at[pl.ds(base_e, E_TILE)], srcl_v.at[pl.ds(0, E_TILE)])
    pltpu.sync_copy(dst.at[pl.ds(base_e, E_TILE)], dstl_v.at[pl.ds(0, E_TILE)])

    def _stage_idx(j, _):
        pltpu.sync_copy(src.at[pl.ds(base_e + j * CH, CH)], src2_v.at[j])
        pltpu.sync_copy(dst.at[pl.ds(base_e + j * CH, CH)], dst2_v.at[j])
        return 0

    lax.fori_loop(0, NCHUNK - 1, _stage_idx, 0)
    pltpu.sync_copy(src.at[pl.ds(base_e + (NCHUNK - 1) * CH, E_LAST)],
                    src2_v.at[NCHUNK - 1, pl.ds(0, E_LAST)])
    pltpu.sync_copy(dst.at[pl.ds(base_e + (NCHUNK - 1) * CH, E_LAST)],
                    dst2_v.at[NCHUNK - 1, pl.ds(0, E_LAST)])
    pltpu.sync_copy(hs, hs_v)
    pltpu.sync_copy(hd, hd_v)
    plsc.subcore_barrier()

    def _vmax(ref):
        def step(i, m):
            return jnp.maximum(m, ref[pl.ds(i * L, L)])
        return jnp.max(lax.fori_loop(0, NP // L, step,
                                     jnp.full((L,), -jnp.inf, _f32)))

    shift = _leaky(_vmax(hs_v) + _vmax(hd_v))

    def _logit_step(i, _):
        off = i * L
        si = srcl_v[pl.ds(off, L)]
        di = dstl_v[pl.ds(off, L)]
        logit = (plsc.load_gather(hs_v, [si]) +
                 plsc.load_gather(hd_v, [di]))
        ex = jnp.exp(_leaky(logit) - shift)
        valid = (off + _iota16()) < E_TILE
        ex_v[pl.ds(off, L)] = jnp.where(valid, ex, 0.0)
        return 0

    lax.fori_loop(0, EP_TILE // L, _logit_step, 0)

    def _den_step(k, _):
        pltpu.sync_copy(ex_v.at[pl.ds(k * CH, CH)],
                        den_sh.at[dst2_v.at[k]], add=True)
        return 0

    lax.fori_loop(0, NCHUNK, _den_step, 0)

    col_iota = [j * L + _iota16() for j in range(HH // L)]

    def _scale_chunk(k, rows):
        def g_step(g, _):
            exbase = k * CH + g * L
            for r in range(L):
                row = jnp.full((L,), g * L + r, dtype=_i32)
                exr = plsc.load_gather(ex_v, [jnp.full((L,), exbase + r, _i32)])
                for j in range(HH // L):
                    v = plsc.load_gather(rows, [row, col_iota[j]])
                    plsc.store_scatter(rows, [row, col_iota[j]], v * exr)
            return 0
        lax.fori_loop(0, CH // L, g_step, 0)

    for p, hp in ((0, h0), (1, h1)):
        pltpu.async_copy(hp.at[src2_v.at[0]], rows_a, sem_a)
        pltpu.async_copy(hp.at[src2_v.at[1]], rows_b, sem_b)

        def _chunk_step(outer, _):
            for b, rows, sem in ((0, rows_a, sem_a), (1, rows_b, sem_b)):
                k = 2 * outer + b
                pltpu.make_async_copy(hp.at[pl.ds(0, CH)], rows, sem).wait()
                _scale_chunk(k, rows)
                pltpu.sync_copy(rows, num_sh.at[dst2_v.at[k]], add=True)

                @pl.when(outer < NCHUNK // 2 - 1)
                def _():
                    pltpu.async_copy(hp.at[src2_v.at[k + 2]], rows, sem)
            return 0

        lax.fori_loop(0, NCHUNK // 2, _chunk_step, 0)
        plsc.subcore_barrier()

        pltpu.sync_copy(num_sh.at[pl.ds(s * 640, 640)],
                        nump.at[c, p, pl.ds(s * 640, 640)])
        if p == 0:
            def _rezero(t, _):
                pltpu.sync_copy(zb2, num_sh.at[pl.ds(s * 640 + t * L, L)])
                return 0
            lax.fori_loop(0, 640 // L, _rezero, 0)
            plsc.subcore_barrier()

    pltpu.sync_copy(den_sh.at[pl.ds(s * 640, 640)],
                    denp.at[c, pl.ds(s * 640, 640)])


_edge = pl.kernel(
    _edge_body,
    out_type=(jax.ShapeDtypeStruct((NC, 2, NP, HH), _f32),
              jax.ShapeDtypeStruct((NC, NP), _f32)),
    mesh=_MESH,
    scratch_types=[
        pltpu.VMEM((NP,), _f32),
        pltpu.VMEM((NP,), _f32),
        pltpu.VMEM((EP_TILE,), _i32),
        pltpu.VMEM((EP_TILE,), _i32),
        pltpu.VMEM((EP_TILE,), _f32),
        pltpu.VMEM((NCHUNK, CH), _i32),
        pltpu.VMEM((NCHUNK, CH), _i32),
        pltpu.VMEM((CH, HH), _f32),
        pltpu.VMEM((CH, HH), _f32),
        pltpu.VMEM((L, HH), _f32),
        pltpu.VMEM((640,), _f32),
        pltpu.VMEM_SHARED((NP, HH), _f32),
        pltpu.VMEM_SHARED((NP,), _f32),
        pltpu.SemaphoreType.DMA,
        pltpu.SemaphoreType.DMA,
    ],
    compiler_params=pltpu.CompilerParams(needs_layout_passes=False, use_tc_tiling_on_sc=False),
)


_BLK = 512
_GRID = NP // _BLK


def _tc_layer_body(x_ref, w_ref, as_ref, ad_ref,
                   h0_ref, h1_ref, hs_ref, hd_ref):
    h = jnp.dot(x_ref[...], w_ref[...], preferred_element_type=_f32)
    h0_ref[...] = h[:, :HH]
    h1_ref[...] = h[:, HH:]
    hs_ref[...] = jnp.dot(h, as_ref[...], preferred_element_type=_f32)
    hd_ref[...] = jnp.dot(h, ad_ref[...], preferred_element_type=_f32)


_H_OUT_SPECS = [
    pl.BlockSpec((_BLK, HH), lambda i: (i, 0)),
    pl.BlockSpec((_BLK, HH), lambda i: (i, 0)),
    pl.BlockSpec((_BLK, 1), lambda i: (i, 0)),
    pl.BlockSpec((_BLK, 1), lambda i: (i, 0)),
]
_H_OUT_SHAPE = [
    jax.ShapeDtypeStruct((NP, HH), _f32),
    jax.ShapeDtypeStruct((NP, HH), _f32),
    jax.ShapeDtypeStruct((NP, 1), _f32),
    jax.ShapeDtypeStruct((NP, 1), _f32),
]
_NUM_SPECS = [
    pl.BlockSpec((1, 1, _BLK, HH), lambda i: (0, 0, i, 0)),
    pl.BlockSpec((1, 1, _BLK, HH), lambda i: (0, 1, i, 0)),
    pl.BlockSpec((1, 1, _BLK, HH), lambda i: (1, 0, i, 0)),
    pl.BlockSpec((1, 1, _BLK, HH), lambda i: (1, 1, i, 0)),
]


def _tc_layer(x, W, a_s, a_d):
    return pl.pallas_call(
        _tc_layer_body,
        grid=(_GRID,),
        in_specs=[
            pl.BlockSpec((_BLK, H), lambda i: (i, 0)),
            pl.BlockSpec((H, H), lambda i: (0, 0)),
            pl.BlockSpec((H, 1), lambda i: (0, 0)),
            pl.BlockSpec((H, 1), lambda i: (0, 0)),
        ],
        out_specs=_H_OUT_SPECS,
        out_shape=_H_OUT_SHAPE,
    )(x, W, a_s, a_d)


def _merged_x(n00, n01, n10, n11, d0, d1, b):
    den = d0[0] + d1[0] + 1e-16
    left = (n00[0, 0] + n10[0, 0]) / den + b[:, :HH]
    right = (n01[0, 0] + n11[0, 0]) / den + b[:, HH:]
    return jnp.concatenate([left, right], axis=1)


def _merge_layer_body(n00_ref, n01_ref, n10_ref, n11_ref, d0_ref, d1_ref,
                      b_ref, w_ref, as_ref, ad_ref,
                      h0_ref, h1_ref, hs_ref, hd_ref):
    xn = _merged_x(n00_ref[...], n01_ref[...], n10_ref[...], n11_ref[...],
                   d0_ref[...], d1_ref[...], b_ref[...])
    xn = jnp.maximum(xn, 0.0)
    h = jnp.dot(xn, w_ref[...], preferred_element_type=_f32)
    h0_ref[...] = h[:, :HH]
    h1_ref[...] = h[:, HH:]
    hs_ref[...] = jnp.dot(h, as_ref[...], preferred_element_type=_f32)
    hd_ref[...] = jnp.dot(h, ad_ref[...], preferred_element_type=_f32)


def _merge_layer(nump, denp, b, W, a_s, a_d):
    return pl.pallas_call(
        _merge_layer_body,
        grid=(_GRID,),
        in_specs=_NUM_SPECS + [
            pl.BlockSpec((1, _BLK, 1), lambda i: (0, i, 0)),
            pl.BlockSpec((1, _BLK, 1), lambda i: (1, i, 0)),
            pl.BlockSpec((1, H), lambda i: (0, 0)),
            pl.BlockSpec((H, H), lambda i: (0, 0)),
            pl.BlockSpec((H, 1), lambda i: (0, 0)),
            pl.BlockSpec((H, 1), lambda i: (0, 0)),
        ],
        out_specs=_H_OUT_SPECS,
        out_shape=_H_OUT_SHAPE,
    )(nump, nump, nump, nump, denp, denp, b, W, a_s, a_d)


_FBLK = 400
_FGRID = N // _FBLK

_FNUM_SPECS = [
    pl.BlockSpec((1, 1, _FBLK, HH), lambda i: (0, 0, i, 0)),
    pl.BlockSpec((1, 1, _FBLK, HH), lambda i: (0, 1, i, 0)),
    pl.BlockSpec((1, 1, _FBLK, HH), lambda i: (1, 0, i, 0)),
    pl.BlockSpec((1, 1, _FBLK, HH), lambda i: (1, 1, i, 0)),
]


def _final_body(n00_ref, n01_ref, n10_ref, n11_ref, d0_ref, d1_ref,
                b_ref, o_ref):
    o_ref[...] = _merged_x(n00_ref[...], n01_ref[...], n10_ref[...],
                           n11_ref[...], d0_ref[...], d1_ref[...], b_ref[...])


def _final_merge(nump, denp, b):
    return pl.pallas_call(
        _final_body,
        grid=(_FGRID,),
        in_specs=_FNUM_SPECS + [
            pl.BlockSpec((1, _FBLK, 1), lambda i: (0, i, 0)),
            pl.BlockSpec((1, _FBLK, 1), lambda i: (1, i, 0)),
            pl.BlockSpec((1, H), lambda i: (0, 0)),
        ],
        out_specs=pl.BlockSpec((_FBLK, H), lambda i: (i, 0)),
        out_shape=jax.ShapeDtypeStruct((N, H), _f32),
    )(nump, nump, nump, nump, denp, denp, b)


@jax.jit
def _run(edge_index, embeddings, Ws, a_src, a_dst, bias):
    flat = edge_index.reshape(-1)
    inv, x0 = _pre(flat, embeddings)
    src, dst = inv[:E], inv[E:]

    h0, h1, hs, hd = _tc_layer(x0, Ws[0], a_src[0][:, None], a_dst[0][:, None])
    nump, denp = _edge(src, dst, hs.reshape(-1), hd.reshape(-1), h0, h1)

    h20, h21, hs2, hd2 = _merge_layer(nump, denp[:, :, None], bias[0][None, :],
                                      Ws[1], a_src[1][:, None], a_dst[1][:, None])
    nump2, denp2 = _edge(src, dst, hs2.reshape(-1), hd2.reshape(-1), h20, h21)

    return _final_merge(nump2, denp2[:, :, None], bias[1][None, :])


def kernel(edge_index, embeddings, Ws, a_src, a_dst, bias):
    return _run(edge_index, embeddings, Ws, a_src, a_dst, bias)


# speedup vs baseline: 10.8092x; 1.2095x over previous
import jax
import jax.numpy as jnp
from jax import lax
from jax.experimental import pallas as pl
from jax.experimental.pallas import tpu as pltpu
from jax.experimental.pallas import tpu_sc as plsc

N = 10000
H = 128
E = 160000
F = 2 * E

NC, NS, L = 2, 16, 16
NW = NC * NS

NP = 10240
UQ = 10496
TRASH = 10240

E_TILE = E // NW
CH = 128
NCHUNK = 40
EP_TILE = NCHUNK * CH
E_LAST = E_TILE - (NCHUNK - 1) * CH

HV = 20000
HROWS = HV // CH
HREM = HV - HROWS * CH

_MESH = plsc.VectorSubcoreMesh(core_axis_name="c", subcore_axis_name="s",
                               num_cores=NC, num_subcores=NS)

_i32 = jnp.int32
_f32 = jnp.float32


def _iota16():
    return lax.iota(_i32, L)


def _leaky(x):
    return jnp.where(x > 0, x, 0.2 * x)


def _splat_lane(vec, r):
    return jnp.broadcast_to(lax.slice(vec, (r,), (r + 1,)), (L,))


def _pre_body(flat, flat3, emb, inv_out, x0_out,
              vals_v, ones_v, cnt_v, rank_v, evals_v, inv_v,
              idx_u, val_u, zb_v, idxrow_v, rows_v,
              cnt_sh, uniq_sh, sem, sem_h):
    s = lax.axis_index("s")
    c = lax.axis_index("c")
    wid = c * NS + s

    for j in range(656 // L):
        zb_v[pl.ds(j * L, L)] = jnp.zeros((L,), _i32)
    for j in range(CH // L):
        ones_v[pl.ds(j * L, L)] = jnp.ones((L,), _i32)
    pltpu.sync_copy(zb_v, cnt_sh.at[pl.ds(s * 656, 656)])
    pltpu.sync_copy(zb_v, uniq_sh.at[pl.ds(s * 656, 656)])
    plsc.subcore_barrier()

    pltpu.sync_copy(flat3.at[s], vals_v)

    def _hist_fire(j, _):
        pltpu.async_copy(ones_v, cnt_sh.at[vals_v.at[j]], sem_h, add=True)
        return 0

    lax.fori_loop(0, HROWS + 1, _hist_fire, 0)

    def _hist_drain(j, _):
        pltpu.make_async_copy(ones_v, cnt_sh.at[vals_v.at[0]], sem_h).wait()
        return 0

    lax.fori_loop(0, HROWS + 1, _hist_drain, 0)
    plsc.subcore_barrier()

    pltpu.sync_copy(cnt_sh.at[pl.ds(0, NP)], cnt_v)

    def _scan_step(i, carry):
        v = cnt_v[pl.ds(i * L, L)]
        b = jnp.where(v > 0, 1, 0).astype(_i32)
        ps = plsc.cumsum(b) + carry
        rank_v[pl.ds(i * L, L)] = ps
        return jnp.max(ps)

    lax.fori_loop(0, NP // L, _scan_step, jnp.int32(0))

    pltpu.sync_copy(flat.at[pl.ds(wid * 10000, 10000)], evals_v)

    def _inv_step(i, _):
        idx = evals_v[pl.ds(i * L, L)]
        inv_v[pl.ds(i * L, L)] = plsc.load_gather(rank_v, [idx]) - 1
        return 0

    lax.fori_loop(0, 10000 // L, _inv_step, 0)
    pltpu.sync_copy(inv_v, inv_out.at[pl.ds(wid * 10000, 10000)])

    base = s * 640
    for i in range(640 // L):
        off = base + i * L
        vv = off + _iota16()
        cntv = cnt_v[pl.ds(off, L)]
        rankv = rank_v[pl.ds(off, L)]
        tgt = jnp.where(cntv > 0, rankv - 1, TRASH)
        idx_u[i // 8, pl.ds((i % 8) * L, L)] = tgt
        val_u[i // 8, pl.ds((i % 8) * L, L)] = vv
    for j in range(5):
        pltpu.sync_copy(val_u.at[j], uniq_sh.at[idx_u.at[j]])
    plsc.subcore_barrier()

    r0 = wid * 320
    for j in range(4):
        pltpu.sync_copy(uniq_sh.at[pl.ds(r0 + j * 80, 80)], idxrow_v.at[j])
    for j in range(4):
        pltpu.async_copy(emb.at[idxrow_v.at[j]],
                         rows_v.at[pl.ds(j * 80, 80)], sem)
    for j in range(4):
        pltpu.make_async_copy(emb.at[idxrow_v.at[0]],
                              rows_v.at[pl.ds(0, 80)], sem).wait()
    pltpu.sync_copy(rows_v, x0_out.at[pl.ds(r0, 320)])


_pre = pl.kernel(
    _pre_body,
    out_type=(jax.ShapeDtypeStruct((F,), _i32),
              jax.ShapeDtypeStruct((NP, H), _f32)),
    mesh=_MESH,
    scratch_types=[
        pltpu.VMEM((HROWS + 1, CH), _i32),
        pltpu.VMEM((CH,), _i32),
        pltpu.VMEM((NP,), _i32),
        pltpu.VMEM((NP,), _i32),
        pltpu.VMEM((10000,), _i32),
        pltpu.VMEM((10000,), _i32),
        pltpu.VMEM((5, CH), _i32),
        pltpu.VMEM((5, CH), _i32),
        pltpu.VMEM((656,), _i32),
        pltpu.VMEM((4, 80), _i32),
        pltpu.VMEM((320, H), _f32),
        pltpu.VMEM_SHARED((UQ,), _i32),
        pltpu.VMEM_SHARED((UQ,), _i32),
        pltpu.SemaphoreType.DMA,
        pltpu.SemaphoreType.DMA,
    ],
    compiler_params=pltpu.CompilerParams(needs_layout_passes=False, use_tc_tiling_on_sc=False),
)


HH = H // 2


def _edge_body(src3, dst3, hs, hd, h0, h1, z2, z1, nump, denp,
               hs_v, hd_v, ex_v, src2_v, dst2_v,
               ga, gb, sa, sb,
               num_sh, den_sh, sem_den, sem_ga, sem_gb, sem_sa, sem_sb):
    s = lax.axis_index("s")
    c = lax.axis_index("c")
    wid = c * NS + s

    pltpu.sync_copy(src3.at[wid], src2_v)
    pltpu.sync_copy(dst3.at[wid], dst2_v)
    pltpu.sync_copy(hs, hs_v)
    pltpu.sync_copy(hd, hd_v)
    pltpu.sync_copy(z2.at[pl.ds(s * 640, 640)], num_sh.at[pl.ds(s * 640, 640)])
    pltpu.sync_copy(z1.at[pl.ds(s * 640, 640)], den_sh.at[pl.ds(s * 640, 640)])
    plsc.subcore_barrier()

    def _vmax(ref):
        def step(i, m):
            return jnp.maximum(m, ref[pl.ds(i * L, L)])
        return jnp.max(lax.fori_loop(0, NP // L, step,
                                     jnp.full((L,), -jnp.inf, _f32)))

    shift = _leaky(_vmax(hs_v) + _vmax(hd_v))

    def _logit_step(i, _):
        k = i // (CH // L)
        g = i % (CH // L)
        si = src2_v[k, pl.ds(g * L, L)]
        di = dst2_v[k, pl.ds(g * L, L)]
        logit = (plsc.load_gather(hs_v, [si]) +
                 plsc.load_gather(hd_v, [di]))
        ex = jnp.exp(_leaky(logit) - shift)
        valid = (i * L + _iota16()) < E_TILE
        ex_v[pl.ds(i * L, L)] = jnp.where(valid, ex, 0.0)
        return 0

    lax.fori_loop(0, EP_TILE // L, _logit_step, 0)

    def _den_fire(k, _):
        pltpu.async_copy(ex_v.at[pl.ds(k * CH, CH)],
                         den_sh.at[dst2_v.at[k]], sem_den, add=True)
        return 0

    lax.fori_loop(0, NCHUNK, _den_fire, 0)

    col_iota = [j * L + _iota16() for j in range(HH // L)]

    def _scale_chunk(k, gsrc, sdst):
        def g_step(g, _):
            exbase = k * CH + g * L
            for r in range(L):
                row = jnp.full((L,), g * L + r, dtype=_i32)
                exr = plsc.load_gather(ex_v, [jnp.full((L,), exbase + r, _i32)])
                for j in range(HH // L):
                    v = plsc.load_gather(gsrc, [row, col_iota[j]])
                    plsc.store_scatter(sdst, [row, col_iota[j]], v * exr)
            return 0
        lax.fori_loop(0, CH // L, g_step, 0)

    for p, hp in ((0, h0), (1, h1)):
        pltpu.async_copy(hp.at[src2_v.at[0]], ga, sem_ga)
        pltpu.async_copy(hp.at[src2_v.at[1]], gb, sem_gb)

        def _chunk_step(outer, _):
            for b, g_buf, s_buf, sg, ss in ((0, ga, sa, sem_ga, sem_sa),
                                            (1, gb, sb, sem_gb, sem_sb)):
                k = 2 * outer + b
                pltpu.make_async_copy(hp.at[pl.ds(0, CH)], g_buf, sg).wait()

                @pl.when(outer > 0)
                def _():
                    pltpu.make_async_copy(s_buf, num_sh.at[dst2_v.at[0]],
                                          ss).wait()

                _scale_chunk(k, g_buf, s_buf)
                pltpu.async_copy(s_buf, num_sh.at[dst2_v.at[k]], ss, add=True)

                @pl.when(outer < NCHUNK // 2 - 1)
                def _():
                    pltpu.async_copy(hp.at[src2_v.at[k + 2]], g_buf, sg)
            return 0

        lax.fori_loop(0, NCHUNK // 2, _chunk_step, 0)
        for s_buf, ss in ((sa, sem_sa), (sb, sem_sb)):
            pltpu.make_async_copy(s_buf, num_sh.at[dst2_v.at[0]], ss).wait()
        if p == 1:
            def _den_drain(k, _):
                pltpu.make_async_copy(ex_v.at[pl.ds(0, CH)],
                                      den_sh.at[dst2_v.at[0]], sem_den).wait()
                return 0
            lax.fori_loop(0, NCHUNK, _den_drain, 0)
        plsc.subcore_barrier()

        pltpu.sync_copy(num_sh.at[pl.ds(s * 640, 640)],
                        nump.at[c, p, pl.ds(s * 640, 640)])
        if p == 0:
            pltpu.sync_copy(z2.at[pl.ds(s * 640, 640)],
                            num_sh.at[pl.ds(s * 640, 640)])
            plsc.subcore_barrier()

    pltpu.sync_copy(den_sh.at[pl.ds(s * 640, 640)],
                    denp.at[c, pl.ds(s * 640, 640)])


_edge = pl.kernel(
    _edge_body,
    out_type=(jax.ShapeDtypeStruct((NC, 2, NP, HH), _f32),
              jax.ShapeDtypeStruct((NC, NP), _f32)),
    mesh=_MESH,
    scratch_types=[
        pltpu.VMEM((NP,), _f32),
        pltpu.VMEM((NP,), _f32),
        pltpu.VMEM((EP_TILE,), _f32),
        pltpu.VMEM((NCHUNK, CH), _i32),
        pltpu.VMEM((NCHUNK, CH), _i32),
        pltpu.VMEM((CH, HH), _f32),
        pltpu.VMEM((CH, HH), _f32),
        pltpu.VMEM((CH, HH), _f32),
        pltpu.VMEM((CH, HH), _f32),
        pltpu.VMEM_SHARED((NP, HH), _f32),
        pltpu.VMEM_SHARED((NP,), _f32),
        pltpu.SemaphoreType.DMA,
        pltpu.SemaphoreType.DMA,
        pltpu.SemaphoreType.DMA,
        pltpu.SemaphoreType.DMA,
        pltpu.SemaphoreType.DMA,
    ],
    compiler_params=pltpu.CompilerParams(needs_layout_passes=False, use_tc_tiling_on_sc=False),
)


_BLK = 512
_GRID = NP // _BLK


def _tc_layer_body(x_ref, w_ref, as_ref, ad_ref,
                   h0_ref, h1_ref, hs_ref, hd_ref):
    h = jnp.dot(x_ref[...], w_ref[...], preferred_element_type=_f32)
    h0_ref[...] = h[:, :HH]
    h1_ref[...] = h[:, HH:]
    hs_ref[...] = jnp.dot(h, as_ref[...], preferred_element_type=_f32)
    hd_ref[...] = jnp.dot(h, ad_ref[...], preferred_element_type=_f32)


_H_OUT_SPECS = [
    pl.BlockSpec((_BLK, HH), lambda i: (i, 0)),
    pl.BlockSpec((_BLK, HH), lambda i: (i, 0)),
    pl.BlockSpec((_BLK, 1), lambda i: (i, 0)),
    pl.BlockSpec((_BLK, 1), lambda i: (i, 0)),
]
_H_OUT_SHAPE = [
    jax.ShapeDtypeStruct((NP, HH), _f32),
    jax.ShapeDtypeStruct((NP, HH), _f32),
    jax.ShapeDtypeStruct((NP, 1), _f32),
    jax.ShapeDtypeStruct((NP, 1), _f32),
]
_NUM_SPECS = [
    pl.BlockSpec((1, 1, _BLK, HH), lambda i: (0, 0, i, 0)),
    pl.BlockSpec((1, 1, _BLK, HH), lambda i: (0, 1, i, 0)),
    pl.BlockSpec((1, 1, _BLK, HH), lambda i: (1, 0, i, 0)),
    pl.BlockSpec((1, 1, _BLK, HH), lambda i: (1, 1, i, 0)),
]


def _tc_layer(x, W, a_s, a_d):
    return pl.pallas_call(
        _tc_layer_body,
        grid=(_GRID,),
        in_specs=[
            pl.BlockSpec((_BLK, H), lambda i: (i, 0)),
            pl.BlockSpec((H, H), lambda i: (0, 0)),
            pl.BlockSpec((H, 1), lambda i: (0, 0)),
            pl.BlockSpec((H, 1), lambda i: (0, 0)),
        ],
        out_specs=_H_OUT_SPECS,
        out_shape=_H_OUT_SHAPE,
    )(x, W, a_s, a_d)


def _merged_x(n00, n01, n10, n11, d0, d1, b):
    den = d0[0] + d1[0] + 1e-16
    left = (n00[0, 0] + n10[0, 0]) / den + b[:, :HH]
    right = (n01[0, 0] + n11[0, 0]) / den + b[:, HH:]
    return jnp.concatenate([left, right], axis=1)


def _merge_layer_body(n00_ref, n01_ref, n10_ref, n11_ref, d0_ref, d1_ref,
                      b_ref, w_ref, as_ref, ad_ref,
                      h0_ref, h1_ref, hs_ref, hd_ref):
    xn = _merged_x(n00_ref[...], n01_ref[...], n10_ref[...], n11_ref[...],
                   d0_ref[...], d1_ref[...], b_ref[...])
    xn = jnp.maximum(xn, 0.0)
    h = jnp.dot(xn, w_ref[...], preferred_element_type=_f32)
    h0_ref[...] = h[:, :HH]
    h1_ref[...] = h[:, HH:]
    hs_ref[...] = jnp.dot(h, as_ref[...], preferred_element_type=_f32)
    hd_ref[...] = jnp.dot(h, ad_ref[...], preferred_element_type=_f32)


def _merge_layer(nump, denp, b, W, a_s, a_d):
    return pl.pallas_call(
        _merge_layer_body,
        grid=(_GRID,),
        in_specs=_NUM_SPECS + [
            pl.BlockSpec((1, _BLK, 1), lambda i: (0, i, 0)),
            pl.BlockSpec((1, _BLK, 1), lambda i: (1, i, 0)),
            pl.BlockSpec((1, H), lambda i: (0, 0)),
            pl.BlockSpec((H, H), lambda i: (0, 0)),
            pl.BlockSpec((H, 1), lambda i: (0, 0)),
            pl.BlockSpec((H, 1), lambda i: (0, 0)),
        ],
        out_specs=_H_OUT_SPECS,
        out_shape=_H_OUT_SHAPE,
    )(nump, nump, nump, nump, denp, denp, b, W, a_s, a_d)


_FBLK = 400
_FGRID = N // _FBLK

_FNUM_SPECS = [
    pl.BlockSpec((1, 1, _FBLK, HH), lambda i: (0, 0, i, 0)),
    pl.BlockSpec((1, 1, _FBLK, HH), lambda i: (0, 1, i, 0)),
    pl.BlockSpec((1, 1, _FBLK, HH), lambda i: (1, 0, i, 0)),
    pl.BlockSpec((1, 1, _FBLK, HH), lambda i: (1, 1, i, 0)),
]


def _final_body(n00_ref, n01_ref, n10_ref, n11_ref, d0_ref, d1_ref,
                b_ref, o_ref):
    o_ref[...] = _merged_x(n00_ref[...], n01_ref[...], n10_ref[...],
                           n11_ref[...], d0_ref[...], d1_ref[...], b_ref[...])


def _final_merge(nump, denp, b):
    return pl.pallas_call(
        _final_body,
        grid=(_FGRID,),
        in_specs=_FNUM_SPECS + [
            pl.BlockSpec((1, _FBLK, 1), lambda i: (0, i, 0)),
            pl.BlockSpec((1, _FBLK, 1), lambda i: (1, i, 0)),
            pl.BlockSpec((1, H), lambda i: (0, 0)),
        ],
        out_specs=pl.BlockSpec((_FBLK, H), lambda i: (i, 0)),
        out_shape=jax.ShapeDtypeStruct((N, H), _f32),
    )(nump, nump, nump, nump, denp, denp, b)


def _pad_edges(idx):
    p = jnp.pad(idx.reshape(NW, E_TILE), ((0, 0), (0, EP_TILE - E_TILE)))
    return p.reshape(NW, NCHUNK, CH)


@jax.jit
def _run(edge_index, embeddings, Ws, a_src, a_dst, bias):
    flat = edge_index.reshape(-1)
    flat3 = jnp.pad(flat.reshape(NS, HV), ((0, 0), (0, CH - HREM)),
                    constant_values=TRASH).reshape(NS, HROWS + 1, CH)
    inv, x0 = _pre(flat, flat3, embeddings)
    src3 = _pad_edges(inv[:E])
    dst3 = _pad_edges(inv[E:])
    z2 = jnp.zeros((NP, HH), _f32)
    z1 = jnp.zeros((NP,), _f32)

    h0, h1, hs, hd = _tc_layer(x0, Ws[0], a_src[0][:, None], a_dst[0][:, None])
    nump, denp = _edge(src3, dst3, hs.reshape(-1),
                       hd.reshape(-1), h0, h1, z2, z1)

    h20, h21, hs2, hd2 = _merge_layer(nump, denp[:, :, None], bias[0][None, :],
                                      Ws[1], a_src[1][:, None], a_dst[1][:, None])
    nump2, denp2 = _edge(src3, dst3, hs2.reshape(-1),
                         hd2.reshape(-1), h20, h21, z2, z1)

    return _final_merge(nump2, denp2[:, :, None], bias[1][None, :])


def kernel(edge_index, embeddings, Ws, a_src, a_dst, bias):
    return _run(edge_index, embeddings, Ws, a_src, a_dst, bias)


# speedup vs baseline: 12.4234x; 1.1493x over previous
import jax
import jax.numpy as jnp
from jax import lax
from jax.experimental import pallas as pl
from jax.experimental.pallas import tpu as pltpu
from jax.experimental.pallas import tpu_sc as plsc

N = 10000
H = 128
E = 160000
F = 2 * E

NC, NS, L = 2, 16, 16
NW = NC * NS

NP = 10240
UQ = 10496
TRASH = 10240

E_TILE = E // NW
CH = 128
NCHUNK = 40
EP_TILE = NCHUNK * CH
E_LAST = E_TILE - (NCHUNK - 1) * CH

HV = 20000
HROWS = HV // CH
HREM = HV - HROWS * CH

_MESH = plsc.VectorSubcoreMesh(core_axis_name="c", subcore_axis_name="s",
                               num_cores=NC, num_subcores=NS)

_i32 = jnp.int32
_f32 = jnp.float32


def _iota16():
    return lax.iota(_i32, L)


def _leaky(x):
    return jnp.where(x > 0, x, 0.2 * x)


def _splat_lane(vec, r):
    return jnp.broadcast_to(lax.slice(vec, (r,), (r + 1,)), (L,))


def _pre_body(flat, flat3, emb, inv_out, x0_out,
              vals_v, ones_v, cnt_v, rank_v, evals_v, inv_v,
              idx_u, val_u, zb_v, idxrow_v, rows_v,
              cnt_sh, uniq_sh, sem, sem_h):
    s = lax.axis_index("s")
    c = lax.axis_index("c")
    wid = c * NS + s

    for j in range(656 // L):
        zb_v[pl.ds(j * L, L)] = jnp.zeros((L,), _i32)
    for j in range(CH // L):
        ones_v[pl.ds(j * L, L)] = jnp.ones((L,), _i32)
    pltpu.sync_copy(zb_v, cnt_sh.at[pl.ds(s * 656, 656)])
    pltpu.sync_copy(zb_v, uniq_sh.at[pl.ds(s * 656, 656)])
    plsc.subcore_barrier()

    pltpu.sync_copy(flat3.at[s], vals_v)

    def _hist_fire(j, _):
        pltpu.async_copy(ones_v, cnt_sh.at[vals_v.at[j]], sem_h, add=True)
        return 0

    lax.fori_loop(0, HROWS + 1, _hist_fire, 0)

    def _hist_drain(j, _):
        pltpu.make_async_copy(ones_v, cnt_sh.at[vals_v.at[0]], sem_h).wait()
        return 0

    lax.fori_loop(0, HROWS + 1, _hist_drain, 0)
    plsc.subcore_barrier()

    pltpu.sync_copy(cnt_sh.at[pl.ds(0, NP)], cnt_v)

    def _scan_step(i, carry):
        v = cnt_v[pl.ds(i * L, L)]
        b = jnp.where(v > 0, 1, 0).astype(_i32)
        ps = plsc.cumsum(b) + carry
        rank_v[pl.ds(i * L, L)] = ps
        return jnp.max(ps)

    lax.fori_loop(0, NP // L, _scan_step, jnp.int32(0))

    pltpu.sync_copy(flat.at[pl.ds(wid * 10000, 10000)], evals_v)

    def _inv_step(i, _):
        idx = evals_v[pl.ds(i * L, L)]
        inv_v[pl.ds(i * L, L)] = plsc.load_gather(rank_v, [idx]) - 1
        return 0

    lax.fori_loop(0, 10000 // L, _inv_step, 0)
    pltpu.sync_copy(inv_v, inv_out.at[pl.ds(wid * 10000, 10000)])

    base = s * 640
    for i in range(640 // L):
        off = base + i * L
        vv = off + _iota16()
        cntv = cnt_v[pl.ds(off, L)]
        rankv = rank_v[pl.ds(off, L)]
        tgt = jnp.where(cntv > 0, rankv - 1, TRASH)
        idx_u[i // 8, pl.ds((i % 8) * L, L)] = tgt
        val_u[i // 8, pl.ds((i % 8) * L, L)] = vv
    for j in range(5):
        pltpu.sync_copy(val_u.at[j], uniq_sh.at[idx_u.at[j]])
    plsc.subcore_barrier()

    r0 = wid * 320
    for j in range(4):
        pltpu.sync_copy(uniq_sh.at[pl.ds(r0 + j * 80, 80)], idxrow_v.at[j])
    for j in range(4):
        pltpu.async_copy(emb.at[idxrow_v.at[j]],
                         rows_v.at[pl.ds(j * 80, 80)], sem)
    for j in range(4):
        pltpu.make_async_copy(emb.at[idxrow_v.at[0]],
                              rows_v.at[pl.ds(0, 80)], sem).wait()
    pltpu.sync_copy(rows_v, x0_out.at[pl.ds(r0, 320)])


_pre = pl.kernel(
    _pre_body,
    out_type=(jax.ShapeDtypeStruct((F,), _i32),
              jax.ShapeDtypeStruct((NP, H), _f32)),
    mesh=_MESH,
    scratch_types=[
        pltpu.VMEM((HROWS + 1, CH), _i32),
        pltpu.VMEM((CH,), _i32),
        pltpu.VMEM((NP,), _i32),
        pltpu.VMEM((NP,), _i32),
        pltpu.VMEM((10000,), _i32),
        pltpu.VMEM((10000,), _i32),
        pltpu.VMEM((5, CH), _i32),
        pltpu.VMEM((5, CH), _i32),
        pltpu.VMEM((656,), _i32),
        pltpu.VMEM((4, 80), _i32),
        pltpu.VMEM((320, H), _f32),
        pltpu.VMEM_SHARED((UQ,), _i32),
        pltpu.VMEM_SHARED((UQ,), _i32),
        pltpu.SemaphoreType.DMA,
        pltpu.SemaphoreType.DMA,
    ],
    compiler_params=pltpu.CompilerParams(needs_layout_passes=False, use_tc_tiling_on_sc=False),
)


HH = H // 2


def _edge_body(src3, dst3, hs, hd, h0, h1, z2, z1, nump, denp,
               hs_v, hd_v, ex_v, src2_v, dst2_v,
               ga, gb, sa, sb,
               num_sh, den_sh, sem_den, sem_ga, sem_gb, sem_sa, sem_sb):
    s = lax.axis_index("s")
    c = lax.axis_index("c")
    wid = c * NS + s

    pltpu.sync_copy(src3.at[wid], src2_v)
    pltpu.sync_copy(dst3.at[wid], dst2_v)
    pltpu.sync_copy(hs, hs_v)
    pltpu.sync_copy(hd, hd_v)
    pltpu.sync_copy(z2.at[pl.ds(s * 640, 640)], num_sh.at[pl.ds(s * 640, 640)])
    pltpu.sync_copy(z1.at[pl.ds(s * 640, 640)], den_sh.at[pl.ds(s * 640, 640)])
    plsc.subcore_barrier()

    def _vmax(ref):
        def step(i, m):
            return jnp.maximum(m, ref[pl.ds(i * L, L)])
        return jnp.max(lax.fori_loop(0, NP // L, step,
                                     jnp.full((L,), -jnp.inf, _f32)))

    shift = _leaky(_vmax(hs_v) + _vmax(hd_v))

    def _logit_step(i, _):
        k = i // (CH // L)
        g = i % (CH // L)
        si = src2_v[k, pl.ds(g * L, L)]
        di = dst2_v[k, pl.ds(g * L, L)]
        logit = (plsc.load_gather(hs_v, [si]) +
                 plsc.load_gather(hd_v, [di]))
        ex = jnp.exp(_leaky(logit) - shift)
        valid = (i * L + _iota16()) < E_TILE
        ex_v[pl.ds(i * L, L)] = jnp.where(valid, ex, 0.0)
        return 0

    lax.fori_loop(0, EP_TILE // L, _logit_step, 0)

    def _den_fire(k, _):
        pltpu.async_copy(ex_v.at[pl.ds(k * CH, CH)],
                         den_sh.at[dst2_v.at[k]], sem_den, add=True)
        return 0

    lax.fori_loop(0, NCHUNK, _den_fire, 0)

    def _scale_chunk(k, gsrc, sdst):
        for row in range(CH):
            exr = plsc.load_gather(ex_v, [jnp.full((L,), k * CH + row, _i32)])
            for j in range(HH // L):
                sdst[row, pl.ds(j * L, L)] = gsrc[row, pl.ds(j * L, L)] * exr

    for p, hp in ((0, h0), (1, h1)):
        pltpu.async_copy(hp.at[src2_v.at[0]], ga, sem_ga)
        pltpu.async_copy(hp.at[src2_v.at[1]], gb, sem_gb)

        def _chunk_step(outer, _):
            for b, g_buf, s_buf, sg, ss in ((0, ga, sa, sem_ga, sem_sa),
                                            (1, gb, sb, sem_gb, sem_sb)):
                k = 2 * outer + b
                pltpu.make_async_copy(hp.at[pl.ds(0, CH)], g_buf, sg).wait()

                @pl.when(outer > 0)
                def _():
                    pltpu.make_async_copy(s_buf, num_sh.at[dst2_v.at[0]],
                                          ss).wait()

                _scale_chunk(k, g_buf, s_buf)
                pltpu.async_copy(s_buf, num_sh.at[dst2_v.at[k]], ss, add=True)

                @pl.when(outer < NCHUNK // 2 - 1)
                def _():
                    pltpu.async_copy(hp.at[src2_v.at[k + 2]], g_buf, sg)
            return 0

        lax.fori_loop(0, NCHUNK // 2, _chunk_step, 0)
        for s_buf, ss in ((sa, sem_sa), (sb, sem_sb)):
            pltpu.make_async_copy(s_buf, num_sh.at[dst2_v.at[0]], ss).wait()
        if p == 1:
            def _den_drain(k, _):
                pltpu.make_async_copy(ex_v.at[pl.ds(0, CH)],
                                      den_sh.at[dst2_v.at[0]], sem_den).wait()
                return 0
            lax.fori_loop(0, NCHUNK, _den_drain, 0)
        plsc.subcore_barrier()

        pltpu.sync_copy(num_sh.at[pl.ds(s * 640, 640)],
                        nump.at[c, p, pl.ds(s * 640, 640)])
        if p == 0:
            pltpu.sync_copy(z2.at[pl.ds(s * 640, 640)],
                            num_sh.at[pl.ds(s * 640, 640)])
            plsc.subcore_barrier()

    pltpu.sync_copy(den_sh.at[pl.ds(s * 640, 640)],
                    denp.at[c, pl.ds(s * 640, 640)])


_edge = pl.kernel(
    _edge_body,
    out_type=(jax.ShapeDtypeStruct((NC, 2, NP, HH), _f32),
              jax.ShapeDtypeStruct((NC, NP), _f32)),
    mesh=_MESH,
    scratch_types=[
        pltpu.VMEM((NP,), _f32),
        pltpu.VMEM((NP,), _f32),
        pltpu.VMEM((EP_TILE,), _f32),
        pltpu.VMEM((NCHUNK, CH), _i32),
        pltpu.VMEM((NCHUNK, CH), _i32),
        pltpu.VMEM((CH, HH), _f32),
        pltpu.VMEM((CH, HH), _f32),
        pltpu.VMEM((CH, HH), _f32),
        pltpu.VMEM((CH, HH), _f32),
        pltpu.VMEM_SHARED((NP, HH), _f32),
        pltpu.VMEM_SHARED((NP,), _f32),
        pltpu.SemaphoreType.DMA,
        pltpu.SemaphoreType.DMA,
        pltpu.SemaphoreType.DMA,
        pltpu.SemaphoreType.DMA,
        pltpu.SemaphoreType.DMA,
    ],
    compiler_params=pltpu.CompilerParams(needs_layout_passes=False, use_tc_tiling_on_sc=False),
)


_BLK = 512
_GRID = NP // _BLK


def _tc_layer_body(x_ref, w_ref, as_ref, ad_ref,
                   h0_ref, h1_ref, hs_ref, hd_ref):
    h = jnp.dot(x_ref[...], w_ref[...], preferred_element_type=_f32)
    h0_ref[...] = h[:, :HH]
    h1_ref[...] = h[:, HH:]
    hs_ref[...] = jnp.dot(h, as_ref[...], preferred_element_type=_f32)
    hd_ref[...] = jnp.dot(h, ad_ref[...], preferred_element_type=_f32)


_H_OUT_SPECS = [
    pl.BlockSpec((_BLK, HH), lambda i: (i, 0)),
    pl.BlockSpec((_BLK, HH), lambda i: (i, 0)),
    pl.BlockSpec((_BLK, 1), lambda i: (i, 0)),
    pl.BlockSpec((_BLK, 1), lambda i: (i, 0)),
]
_H_OUT_SHAPE = [
    jax.ShapeDtypeStruct((NP, HH), _f32),
    jax.ShapeDtypeStruct((NP, HH), _f32),
    jax.ShapeDtypeStruct((NP, 1), _f32),
    jax.ShapeDtypeStruct((NP, 1), _f32),
]
_NUM_SPECS = [
    pl.BlockSpec((1, 1, _BLK, HH), lambda i: (0, 0, i, 0)),
    pl.BlockSpec((1, 1, _BLK, HH), lambda i: (0, 1, i, 0)),
    pl.BlockSpec((1, 1, _BLK, HH), lambda i: (1, 0, i, 0)),
    pl.BlockSpec((1, 1, _BLK, HH), lambda i: (1, 1, i, 0)),
]


def _tc_layer(x, W, a_s, a_d):
    return pl.pallas_call(
        _tc_layer_body,
        grid=(_GRID,),
        in_specs=[
            pl.BlockSpec((_BLK, H), lambda i: (i, 0)),
            pl.BlockSpec((H, H), lambda i: (0, 0)),
            pl.BlockSpec((H, 1), lambda i: (0, 0)),
            pl.BlockSpec((H, 1), lambda i: (0, 0)),
        ],
        out_specs=_H_OUT_SPECS,
        out_shape=_H_OUT_SHAPE,
    )(x, W, a_s, a_d)


def _merged_x(n00, n01, n10, n11, d0, d1, b):
    den = d0[0] + d1[0] + 1e-16
    left = (n00[0, 0] + n10[0, 0]) / den + b[:, :HH]
    right = (n01[0, 0] + n11[0, 0]) / den + b[:, HH:]
    return jnp.concatenate([left, right], axis=1)


def _merge_layer_body(n00_ref, n01_ref, n10_ref, n11_ref, d0_ref, d1_ref,
                      b_ref, w_ref, as_ref, ad_ref,
                      h0_ref, h1_ref, hs_ref, hd_ref):
    xn = _merged_x(n00_ref[...], n01_ref[...], n10_ref[...], n11_ref[...],
                   d0_ref[...], d1_ref[...], b_ref[...])
    xn = jnp.maximum(xn, 0.0)
    h = jnp.dot(xn, w_ref[...], preferred_element_type=_f32)
    h0_ref[...] = h[:, :HH]
    h1_ref[...] = h[:, HH:]
    hs_ref[...] = jnp.dot(h, as_ref[...], preferred_element_type=_f32)
    hd_ref[...] = jnp.dot(h, ad_ref[...], preferred_element_type=_f32)


def _merge_layer(nump, denp, b, W, a_s, a_d):
    return pl.pallas_call(
        _merge_layer_body,
        grid=(_GRID,),
        in_specs=_NUM_SPECS + [
            pl.BlockSpec((1, _BLK, 1), lambda i: (0, i, 0)),
            pl.BlockSpec((1, _BLK, 1), lambda i: (1, i, 0)),
            pl.BlockSpec((1, H), lambda i: (0, 0)),
            pl.BlockSpec((H, H), lambda i: (0, 0)),
            pl.BlockSpec((H, 1), lambda i: (0, 0)),
            pl.BlockSpec((H, 1), lambda i: (0, 0)),
        ],
        out_specs=_H_OUT_SPECS,
        out_shape=_H_OUT_SHAPE,
    )(nump, nump, nump, nump, denp, denp, b, W, a_s, a_d)


_FBLK = 400
_FGRID = N // _FBLK

_FNUM_SPECS = [
    pl.BlockSpec((1, 1, _FBLK, HH), lambda i: (0, 0, i, 0)),
    pl.BlockSpec((1, 1, _FBLK, HH), lambda i: (0, 1, i, 0)),
    pl.BlockSpec((1, 1, _FBLK, HH), lambda i: (1, 0, i, 0)),
    pl.BlockSpec((1, 1, _FBLK, HH), lambda i: (1, 1, i, 0)),
]


def _final_body(n00_ref, n01_ref, n10_ref, n11_ref, d0_ref, d1_ref,
                b_ref, o_ref):
    o_ref[...] = _merged_x(n00_ref[...], n01_ref[...], n10_ref[...],
                           n11_ref[...], d0_ref[...], d1_ref[...], b_ref[...])


def _final_merge(nump, denp, b):
    return pl.pallas_call(
        _final_body,
        grid=(_FGRID,),
        in_specs=_FNUM_SPECS + [
            pl.BlockSpec((1, _FBLK, 1), lambda i: (0, i, 0)),
            pl.BlockSpec((1, _FBLK, 1), lambda i: (1, i, 0)),
            pl.BlockSpec((1, H), lambda i: (0, 0)),
        ],
        out_specs=pl.BlockSpec((_FBLK, H), lambda i: (i, 0)),
        out_shape=jax.ShapeDtypeStruct((N, H), _f32),
    )(nump, nump, nump, nump, denp, denp, b)


def _pad_edges(idx):
    p = jnp.pad(idx.reshape(NW, E_TILE), ((0, 0), (0, EP_TILE - E_TILE)))
    return p.reshape(NW, NCHUNK, CH)


@jax.jit
def _run(edge_index, embeddings, Ws, a_src, a_dst, bias):
    flat = edge_index.reshape(-1)
    flat3 = jnp.pad(flat.reshape(NS, HV), ((0, 0), (0, CH - HREM)),
                    constant_values=TRASH).reshape(NS, HROWS + 1, CH)
    inv, x0 = _pre(flat, flat3, embeddings)
    src3 = _pad_edges(inv[:E])
    dst3 = _pad_edges(inv[E:])
    z2 = jnp.zeros((NP, HH), _f32)
    z1 = jnp.zeros((NP,), _f32)

    h0, h1, hs, hd = _tc_layer(x0, Ws[0], a_src[0][:, None], a_dst[0][:, None])
    nump, denp = _edge(src3, dst3, hs.reshape(-1),
                       hd.reshape(-1), h0, h1, z2, z1)

    h20, h21, hs2, hd2 = _merge_layer(nump, denp[:, :, None], bias[0][None, :],
                                      Ws[1], a_src[1][:, None], a_dst[1][:, None])
    nump2, denp2 = _edge(src3, dst3, hs2.reshape(-1),
                         hd2.reshape(-1), h20, h21, z2, z1)

    return _final_merge(nump2, denp2[:, :, None], bias[1][None, :])


def kernel(edge_index, embeddings, Ws, a_src, a_dst, bias):
    return _run(edge_index, embeddings, Ws, a_src, a_dst, bias)


# speedup vs baseline: 12.4282x; 1.0004x over previous
import jax
import jax.numpy as jnp
from jax import lax
from jax.experimental import pallas as pl
from jax.experimental.pallas import tpu as pltpu
from jax.experimental.pallas import tpu_sc as plsc

N = 10000
H = 128
E = 160000
F = 2 * E

NC, NS, L = 2, 16, 16
NW = NC * NS

NP = 10240
UQ = 10496
TRASH = 10240

E_TILE = E // NW
CH = 128
NCHUNK = 40
EP_TILE = NCHUNK * CH
E_LAST = E_TILE - (NCHUNK - 1) * CH

HV = 20000
HROWS = HV // CH
HREM = HV - HROWS * CH

_MESH = plsc.VectorSubcoreMesh(core_axis_name="c", subcore_axis_name="s",
                               num_cores=NC, num_subcores=NS)

_i32 = jnp.int32
_f32 = jnp.float32


def _iota16():
    return lax.iota(_i32, L)


def _leaky(x):
    return jnp.where(x > 0, x, 0.2 * x)


def _splat_lane(vec, r):
    return jnp.broadcast_to(lax.slice(vec, (r,), (r + 1,)), (L,))


def _pre_body(flat, flat3, emb, inv_out, x0_out,
              vals_v, ones_v, cnt_v, rank_v, evals_v, inv_v,
              idx_u, val_u, zb_v, idxrow_v, rows_v,
              cnt_sh, uniq_sh, sem, sem_h):
    s = lax.axis_index("s")
    c = lax.axis_index("c")
    wid = c * NS + s

    for j in range(656 // L):
        zb_v[pl.ds(j * L, L)] = jnp.zeros((L,), _i32)
    for j in range(CH // L):
        ones_v[pl.ds(j * L, L)] = jnp.ones((L,), _i32)
    pltpu.sync_copy(zb_v, cnt_sh.at[pl.ds(s * 656, 656)])
    pltpu.sync_copy(zb_v, uniq_sh.at[pl.ds(s * 656, 656)])
    plsc.subcore_barrier()

    pltpu.sync_copy(flat3.at[s], vals_v)

    def _hist_fire(j, _):
        pltpu.async_copy(ones_v, cnt_sh.at[vals_v.at[j]], sem_h, add=True)
        return 0

    lax.fori_loop(0, HROWS + 1, _hist_fire, 0)

    def _hist_drain(j, _):
        pltpu.make_async_copy(ones_v, cnt_sh.at[vals_v.at[0]], sem_h).wait()
        return 0

    lax.fori_loop(0, HROWS + 1, _hist_drain, 0)
    plsc.subcore_barrier()

    pltpu.sync_copy(cnt_sh.at[pl.ds(0, NP)], cnt_v)

    def _scan_step(i, carry):
        v = cnt_v[pl.ds(i * L, L)]
        b = jnp.where(v > 0, 1, 0).astype(_i32)
        ps = plsc.cumsum(b) + carry
        rank_v[pl.ds(i * L, L)] = ps
        return jnp.max(ps)

    lax.fori_loop(0, NP // L, _scan_step, jnp.int32(0))

    pltpu.sync_copy(flat.at[pl.ds(wid * 10000, 10000)], evals_v)

    def _inv_step(i, _):
        idx = evals_v[pl.ds(i * L, L)]
        inv_v[pl.ds(i * L, L)] = plsc.load_gather(rank_v, [idx]) - 1
        return 0

    lax.fori_loop(0, 10000 // L, _inv_step, 0)
    pltpu.sync_copy(inv_v, inv_out.at[pl.ds(wid * 10000, 10000)])

    base = s * 640
    for i in range(640 // L):
        off = base + i * L
        vv = off + _iota16()
        cntv = cnt_v[pl.ds(off, L)]
        rankv = rank_v[pl.ds(off, L)]
        tgt = jnp.where(cntv > 0, rankv - 1, TRASH)
        idx_u[i // 8, pl.ds((i % 8) * L, L)] = tgt
        val_u[i // 8, pl.ds((i % 8) * L, L)] = vv
    for j in range(5):
        pltpu.sync_copy(val_u.at[j], uniq_sh.at[idx_u.at[j]])
    plsc.subcore_barrier()

    r0 = wid * 320
    for j in range(4):
        pltpu.sync_copy(uniq_sh.at[pl.ds(r0 + j * 80, 80)], idxrow_v.at[j])
    for j in range(4):
        pltpu.async_copy(emb.at[idxrow_v.at[j]],
                         rows_v.at[pl.ds(j * 80, 80)], sem)
    for j in range(4):
        pltpu.make_async_copy(emb.at[idxrow_v.at[0]],
                              rows_v.at[pl.ds(0, 80)], sem).wait()
    pltpu.sync_copy(rows_v, x0_out.at[pl.ds(r0, 320)])


_pre = pl.kernel(
    _pre_body,
    out_type=(jax.ShapeDtypeStruct((F,), _i32),
              jax.ShapeDtypeStruct((NP, H), _f32)),
    mesh=_MESH,
    scratch_types=[
        pltpu.VMEM((HROWS + 1, CH), _i32),
        pltpu.VMEM((CH,), _i32),
        pltpu.VMEM((NP,), _i32),
        pltpu.VMEM((NP,), _i32),
        pltpu.VMEM((10000,), _i32),
        pltpu.VMEM((10000,), _i32),
        pltpu.VMEM((5, CH), _i32),
        pltpu.VMEM((5, CH), _i32),
        pltpu.VMEM((656,), _i32),
        pltpu.VMEM((4, 80), _i32),
        pltpu.VMEM((320, H), _f32),
        pltpu.VMEM_SHARED((UQ,), _i32),
        pltpu.VMEM_SHARED((UQ,), _i32),
        pltpu.SemaphoreType.DMA,
        pltpu.SemaphoreType.DMA,
    ],
    compiler_params=pltpu.CompilerParams(needs_layout_passes=False, use_tc_tiling_on_sc=False),
)


HH = H // 2


def _edge_body(src3, dst3, hs, hd, h0, h1, z2, z1, nump, denp,
               hs_v, hd_v, ex_v, src2_v, dst2_v,
               ga, gb, sa, sb,
               num_sh, den_sh, sem_den, sem_ga, sem_gb, sem_sa, sem_sb):
    s = lax.axis_index("s")
    c = lax.axis_index("c")
    wid = c * NS + s

    with jax.named_scope("edge_stage"):
        pltpu.sync_copy(src3.at[wid], src2_v)
        pltpu.sync_copy(dst3.at[wid], dst2_v)
        pltpu.sync_copy(hs, hs_v)
        pltpu.sync_copy(hd, hd_v)
        pltpu.sync_copy(z2.at[pl.ds(s * 640, 640)],
                        num_sh.at[pl.ds(s * 640, 640)])
        pltpu.sync_copy(z1.at[pl.ds(s * 640, 640)],
                        den_sh.at[pl.ds(s * 640, 640)])
        plsc.subcore_barrier()

    def _vmax(ref):
        def step(i, m):
            return jnp.maximum(m, ref[pl.ds(i * L, L)])
        return jnp.max(lax.fori_loop(0, NP // L, step,
                                     jnp.full((L,), -jnp.inf, _f32)))

    shift = _leaky(_vmax(hs_v) + _vmax(hd_v))

    def _logit_step(i, _):
        k = i // (CH // L)
        g = i % (CH // L)
        si = src2_v[k, pl.ds(g * L, L)]
        di = dst2_v[k, pl.ds(g * L, L)]
        logit = (plsc.load_gather(hs_v, [si]) +
                 plsc.load_gather(hd_v, [di]))
        ex = jnp.exp(_leaky(logit) - shift)
        valid = (i * L + _iota16()) < E_TILE
        ex_v[pl.ds(i * L, L)] = jnp.where(valid, ex, 0.0)
        return 0

    with jax.named_scope("edge_logit"):
        lax.fori_loop(0, EP_TILE // L, _logit_step, 0)

    def _den_fire(k, _):
        pltpu.async_copy(ex_v.at[pl.ds(k * CH, CH)],
                         den_sh.at[dst2_v.at[k]], sem_den, add=True)
        return 0

    with jax.named_scope("edge_denfire"):
        lax.fori_loop(0, NCHUNK, _den_fire, 0)

    def _scale_chunk(k, gsrc, sdst):
        for row in range(CH):
            exr = plsc.load_gather(ex_v, [jnp.full((L,), k * CH + row, _i32)])
            for j in range(HH // L):
                sdst[row, pl.ds(j * L, L)] = gsrc[row, pl.ds(j * L, L)] * exr

    for p, hp in ((0, h0), (1, h1)):
      with jax.named_scope(f"edge_numpass{p}"):
        pltpu.async_copy(hp.at[src2_v.at[0]], ga, sem_ga)
        pltpu.async_copy(hp.at[src2_v.at[1]], gb, sem_gb)

        def _chunk_step(outer, _):
            for b, g_buf, s_buf, sg, ss in ((0, ga, sa, sem_ga, sem_sa),
                                            (1, gb, sb, sem_gb, sem_sb)):
                k = 2 * outer + b
                pltpu.make_async_copy(hp.at[pl.ds(0, CH)], g_buf, sg).wait()

                @pl.when(outer > 0)
                def _():
                    pltpu.make_async_copy(s_buf, num_sh.at[dst2_v.at[0]],
                                          ss).wait()

                _scale_chunk(k, g_buf, s_buf)
                pltpu.async_copy(s_buf, num_sh.at[dst2_v.at[k]], ss, add=True)

                @pl.when(outer < NCHUNK // 2 - 1)
                def _():
                    pltpu.async_copy(hp.at[src2_v.at[k + 2]], g_buf, sg)
            return 0

        lax.fori_loop(0, NCHUNK // 2, _chunk_step, 0)
        for s_buf, ss in ((sa, sem_sa), (sb, sem_sb)):
            pltpu.make_async_copy(s_buf, num_sh.at[dst2_v.at[0]], ss).wait()
        if p == 1:
            def _den_drain(k, _):
                pltpu.make_async_copy(ex_v.at[pl.ds(0, CH)],
                                      den_sh.at[dst2_v.at[0]], sem_den).wait()
                return 0
            lax.fori_loop(0, NCHUNK, _den_drain, 0)
        plsc.subcore_barrier()

        pltpu.sync_copy(num_sh.at[pl.ds(s * 640, 640)],
                        nump.at[c, p, pl.ds(s * 640, 640)])
        if p == 0:
            pltpu.sync_copy(z2.at[pl.ds(s * 640, 640)],
                            num_sh.at[pl.ds(s * 640, 640)])
            plsc.subcore_barrier()

    pltpu.sync_copy(den_sh.at[pl.ds(s * 640, 640)],
                    denp.at[c, pl.ds(s * 640, 640)])


_edge = pl.kernel(
    _edge_body,
    out_type=(jax.ShapeDtypeStruct((NC, 2, NP, HH), _f32),
              jax.ShapeDtypeStruct((NC, NP), _f32)),
    mesh=_MESH,
    scratch_types=[
        pltpu.VMEM((NP,), _f32),
        pltpu.VMEM((NP,), _f32),
        pltpu.VMEM((EP_TILE,), _f32),
        pltpu.VMEM((NCHUNK, CH), _i32),
        pltpu.VMEM((NCHUNK, CH), _i32),
        pltpu.VMEM((CH, HH), _f32),
        pltpu.VMEM((CH, HH), _f32),
        pltpu.VMEM((CH, HH), _f32),
        pltpu.VMEM((CH, HH), _f32),
        pltpu.VMEM_SHARED((NP, HH), _f32),
        pltpu.VMEM_SHARED((NP,), _f32),
        pltpu.SemaphoreType.DMA,
        pltpu.SemaphoreType.DMA,
        pltpu.SemaphoreType.DMA,
        pltpu.SemaphoreType.DMA,
        pltpu.SemaphoreType.DMA,
    ],
    compiler_params=pltpu.CompilerParams(needs_layout_passes=False, use_tc_tiling_on_sc=False),
)


_BLK = 512
_GRID = NP // _BLK


def _tc_layer_body(x_ref, w_ref, as_ref, ad_ref,
                   h0_ref, h1_ref, hs_ref, hd_ref):
    h = jnp.dot(x_ref[...], w_ref[...], preferred_element_type=_f32)
    h0_ref[...] = h[:, :HH]
    h1_ref[...] = h[:, HH:]
    hs_ref[...] = jnp.dot(h, as_ref[...], preferred_element_type=_f32)
    hd_ref[...] = jnp.dot(h, ad_ref[...], preferred_element_type=_f32)


_H_OUT_SPECS = [
    pl.BlockSpec((_BLK, HH), lambda i: (i, 0)),
    pl.BlockSpec((_BLK, HH), lambda i: (i, 0)),
    pl.BlockSpec((_BLK, 1), lambda i: (i, 0)),
    pl.BlockSpec((_BLK, 1), lambda i: (i, 0)),
]
_H_OUT_SHAPE = [
    jax.ShapeDtypeStruct((NP, HH), _f32),
    jax.ShapeDtypeStruct((NP, HH), _f32),
    jax.ShapeDtypeStruct((NP, 1), _f32),
    jax.ShapeDtypeStruct((NP, 1), _f32),
]
_NUM_SPECS = [
    pl.BlockSpec((1, 1, _BLK, HH), lambda i: (0, 0, i, 0)),
    pl.BlockSpec((1, 1, _BLK, HH), lambda i: (0, 1, i, 0)),
    pl.BlockSpec((1, 1, _BLK, HH), lambda i: (1, 0, i, 0)),
    pl.BlockSpec((1, 1, _BLK, HH), lambda i: (1, 1, i, 0)),
]


def _tc_layer(x, W, a_s, a_d):
    return pl.pallas_call(
        _tc_layer_body,
        grid=(_GRID,),
        in_specs=[
            pl.BlockSpec((_BLK, H), lambda i: (i, 0)),
            pl.BlockSpec((H, H), lambda i: (0, 0)),
            pl.BlockSpec((H, 1), lambda i: (0, 0)),
            pl.BlockSpec((H, 1), lambda i: (0, 0)),
        ],
        out_specs=_H_OUT_SPECS,
        out_shape=_H_OUT_SHAPE,
    )(x, W, a_s, a_d)


def _merged_x(n00, n01, n10, n11, d0, d1, b):
    den = d0[0] + d1[0] + 1e-16
    left = (n00[0, 0] + n10[0, 0]) / den + b[:, :HH]
    right = (n01[0, 0] + n11[0, 0]) / den + b[:, HH:]
    return jnp.concatenate([left, right], axis=1)


def _merge_layer_body(n00_ref, n01_ref, n10_ref, n11_ref, d0_ref, d1_ref,
                      b_ref, w_ref, as_ref, ad_ref,
                      h0_ref, h1_ref, hs_ref, hd_ref):
    xn = _merged_x(n00_ref[...], n01_ref[...], n10_ref[...], n11_ref[...],
                   d0_ref[...], d1_ref[...], b_ref[...])
    xn = jnp.maximum(xn, 0.0)
    h = jnp.dot(xn, w_ref[...], preferred_element_type=_f32)
    h0_ref[...] = h[:, :HH]
    h1_ref[...] = h[:, HH:]
    hs_ref[...] = jnp.dot(h, as_ref[...], preferred_element_type=_f32)
    hd_ref[...] = jnp.dot(h, ad_ref[...], preferred_element_type=_f32)


def _merge_layer(nump, denp, b, W, a_s, a_d):
    return pl.pallas_call(
        _merge_layer_body,
        grid=(_GRID,),
        in_specs=_NUM_SPECS + [
            pl.BlockSpec((1, _BLK, 1), lambda i: (0, i, 0)),
            pl.BlockSpec((1, _BLK, 1), lambda i: (1, i, 0)),
            pl.BlockSpec((1, H), lambda i: (0, 0)),
            pl.BlockSpec((H, H), lambda i: (0, 0)),
            pl.BlockSpec((H, 1), lambda i: (0, 0)),
            pl.BlockSpec((H, 1), lambda i: (0, 0)),
        ],
        out_specs=_H_OUT_SPECS,
        out_shape=_H_OUT_SHAPE,
    )(nump, nump, nump, nump, denp, denp, b, W, a_s, a_d)


_FBLK = 400
_FGRID = N // _FBLK

_FNUM_SPECS = [
    pl.BlockSpec((1, 1, _FBLK, HH), lambda i: (0, 0, i, 0)),
    pl.BlockSpec((1, 1, _FBLK, HH), lambda i: (0, 1, i, 0)),
    pl.BlockSpec((1, 1, _FBLK, HH), lambda i: (1, 0, i, 0)),
    pl.BlockSpec((1, 1, _FBLK, HH), lambda i: (1, 1, i, 0)),
]


def _final_body(n00_ref, n01_ref, n10_ref, n11_ref, d0_ref, d1_ref,
                b_ref, o_ref):
    o_ref[...] = _merged_x(n00_ref[...], n01_ref[...], n10_ref[...],
                           n11_ref[...], d0_ref[...], d1_ref[...], b_ref[...])


def _final_merge(nump, denp, b):
    return pl.pallas_call(
        _final_body,
        grid=(_FGRID,),
        in_specs=_FNUM_SPECS + [
            pl.BlockSpec((1, _FBLK, 1), lambda i: (0, i, 0)),
            pl.BlockSpec((1, _FBLK, 1), lambda i: (1, i, 0)),
            pl.BlockSpec((1, H), lambda i: (0, 0)),
        ],
        out_specs=pl.BlockSpec((_FBLK, H), lambda i: (i, 0)),
        out_shape=jax.ShapeDtypeStruct((N, H), _f32),
    )(nump, nump, nump, nump, denp, denp, b)


def _pad_edges(idx):
    p = jnp.pad(idx.reshape(NW, E_TILE), ((0, 0), (0, EP_TILE - E_TILE)))
    return p.reshape(NW, NCHUNK, CH)


@jax.jit
def _run(edge_index, embeddings, Ws, a_src, a_dst, bias):
    flat = edge_index.reshape(-1)
    flat3 = jnp.pad(flat.reshape(NS, HV), ((0, 0), (0, CH - HREM)),
                    constant_values=TRASH).reshape(NS, HROWS + 1, CH)
    inv, x0 = _pre(flat, flat3, embeddings)
    src3 = _pad_edges(inv[:E])
    dst3 = _pad_edges(inv[E:])
    z2 = jnp.zeros((NP, HH), _f32)
    z1 = jnp.zeros((NP,), _f32)

    h0, h1, hs, hd = _tc_layer(x0, Ws[0], a_src[0][:, None], a_dst[0][:, None])
    nump, denp = _edge(src3, dst3, hs.reshape(-1),
                       hd.reshape(-1), h0, h1, z2, z1)

    h20, h21, hs2, hd2 = _merge_layer(nump, denp[:, :, None], bias[0][None, :],
                                      Ws[1], a_src[1][:, None], a_dst[1][:, None])
    nump2, denp2 = _edge(src3, dst3, hs2.reshape(-1),
                         hd2.reshape(-1), h20, h21, z2, z1)

    return _final_merge(nump2, denp2[:, :, None], bias[1][None, :])


def kernel(edge_index, embeddings, Ws, a_src, a_dst, bias):
    return _run(edge_index, embeddings, Ws, a_src, a_dst, bias)
